# baseline (speedup 1.0000x reference)
import jax
import jax.numpy as jnp
from jax.experimental import pallas as pl
from jax.experimental.pallas import tpu as pltpu

N = 256
NOUT = 128
HEADS = 4
HC1 = 256
HC2 = 128
DBLK = 64


def _attn(xl, xr, att_ref, nd, hc):
    outs = []
    for h in range(HEADS):
        xl_h = xl[:, h * hc:(h + 1) * hc]
        att4 = 0.4 * att_ref[h:h + 1, :]
        xlTf = jnp.transpose(xl_h * att4)
        xlT = xlTf.astype(jnp.bfloat16)
        sgn = jnp.broadcast_to(jnp.sign(jnp.transpose(att4)), (hc, N)).astype(jnp.bfloat16)
        al = jnp.sum(xlTf, axis=0, keepdims=True)
        xr_h = xr[:, h * hc:(h + 1) * hc] * att4
        ar = jnp.sum(xr_h, axis=1, keepdims=True)
        xr16 = xr_h.astype(jnp.bfloat16)

        blocks = []
        for b in range(nd // DBLK):
            d0 = b * DBLK
            xrb = xr16[d0:d0 + DBLK]
            t = xrb[:, :, None] + xlT[None, :, :]
            sc = jnp.sum(jnp.abs(t) * sgn[None, :, :], axis=1,
                         dtype=jnp.bfloat16).astype(jnp.float32)
            sc = sc + 1.5 * (ar[d0:d0 + DBLK] + al)
            rows = d0 + jax.lax.broadcasted_iota(jnp.int32, (DBLK, N), 0)
            cols = jax.lax.broadcasted_iota(jnp.int32, (DBLK, N), 1)
            sc = jnp.where(rows == cols, -1e30, sc)
            m = jnp.max(sc, axis=1, keepdims=True)
            e = jnp.exp(sc - m)
            den = jnp.sum(e, axis=1, keepdims=True)
            a = e / (den + 1e-16)
            blocks.append(jnp.dot(a.astype(jnp.bfloat16), xl_h.astype(jnp.bfloat16),
                                  preferred_element_type=jnp.float32))
        outs.append(jnp.concatenate(blocks, axis=0) if len(blocks) > 1 else blocks[0])
    return jnp.concatenate(outs, axis=1)


def _body(x_ref, wlt1_ref, bl1_ref, wrt1_ref, br1_ref, att1_ref, bias1_ref,
          wlt2_ref, bl2_ref, wrt2_ref, br2_ref, att2_ref, bias2_ref,
          wot_ref, bo_ref, lng_ref, lnb_ref, resw_ref, out_ref):
    x = x_ref[...]
    xb = x.astype(jnp.bfloat16)
    xl1 = jnp.dot(xb, wlt1_ref[...], preferred_element_type=jnp.float32) + bl1_ref[...]
    xr1 = jnp.dot(xb, wrt1_ref[...], preferred_element_type=jnp.float32) + br1_ref[...]
    o1 = _attn(xl1, xr1, att1_ref, N, HC1) + bias1_ref[...]
    y = jnp.where(o1 > 0, o1, jnp.exp(jnp.minimum(o1, 0.0)) - 1.0)

    yb = y.astype(jnp.bfloat16)
    xl2 = jnp.dot(yb, wlt2_ref[...], preferred_element_type=jnp.float32) + bl2_ref[...]
    xr2 = jnp.dot(yb[:NOUT], wrt2_ref[...], preferred_element_type=jnp.float32) + br2_ref[...]
    o2 = _attn(xl2, xr2, att2_ref, NOUT, HC2) + bias2_ref[...]

    h = jnp.dot(o2, wot_ref[...], preferred_element_type=jnp.float32) + bo_ref[...]
    mu = jnp.mean(h, axis=-1, keepdims=True)
    var = jnp.mean((h - mu) ** 2, axis=-1, keepdims=True)
    h = (h - mu) * jax.lax.rsqrt(var + 1e-5) * lng_ref[...] + lnb_ref[...]
    h = jnp.maximum(h, 0.0)
    out_ref[...] = h + resw_ref[0, 0] * x[:NOUT]


def kernel(video_1_fea, video_2_fea, video_1_fused, video_2_fused,
           Wl1, bl1, Wr1, br1, att1, bias1,
           Wl2, bl2, Wr2, br2, att2, bias2,
           Wo, bo, ln_g, ln_b, res_w):
    B1, T, C = video_1_fea.shape
    B2 = video_2_fea.shape[0]
    x = jnp.concatenate([
        video_1_fea.reshape(B1 * T, C),
        video_2_fea.reshape(B2 * T, C),
        video_1_fused.reshape(B1 * T, C),
        video_2_fused.reshape(B2 * T, C),
    ], axis=0)

    f32 = jnp.float32
    h = pl.pallas_call(
        _body,
        out_shape=jax.ShapeDtypeStruct((NOUT, C), f32),
    )(x, Wl1.T.astype(jnp.bfloat16), bl1.reshape(1, -1),
      Wr1.T.astype(jnp.bfloat16), br1.reshape(1, -1), att1, bias1.reshape(1, -1),
      Wl2.T.astype(jnp.bfloat16), bl2.reshape(1, -1),
      Wr2.T.astype(jnp.bfloat16), br2.reshape(1, -1), att2, bias2.reshape(1, -1),
      Wo.T, bo.reshape(1, -1), ln_g.reshape(1, -1), ln_b.reshape(1, -1),
      res_w.reshape(1, 1))

    p1 = h[:B1 * T].reshape(B1, T, C)
    p2 = h[B1 * T:].reshape(B2, T, C)
    return jnp.concatenate([p1, p2], axis=0)


# speedup vs baseline: 1.0147x; 1.0147x over previous
import jax
import jax.numpy as jnp
from jax.experimental import pallas as pl
from jax.experimental.pallas import tpu as pltpu

N = 256
NOUT = 128
HEADS = 4
HC1 = 256
HC2 = 128
DBLK = 32


def _attn(xl, xr, att_ref, nd, hc):
    outs = []
    for h in range(HEADS):
        xl_h = xl[:, h * hc:(h + 1) * hc]
        att4 = 0.4 * att_ref[h:h + 1, :]
        xlTf = jnp.transpose(xl_h * att4)
        xlT = xlTf.astype(jnp.bfloat16)
        sgn = jnp.broadcast_to(jnp.sign(jnp.transpose(att4)), (hc, N)).astype(jnp.bfloat16)
        al = jnp.sum(xlTf, axis=0, keepdims=True)
        xr_h = xr[:, h * hc:(h + 1) * hc] * att4
        ar = jnp.sum(xr_h, axis=1, keepdims=True)
        xr16 = xr_h.astype(jnp.bfloat16)

        blocks = []
        for b in range(nd // DBLK):
            d0 = b * DBLK
            xrb = xr16[d0:d0 + DBLK]
            t = xrb[:, :, None] + xlT[None, :, :]
            sc = jnp.sum(jnp.abs(t) * sgn[None, :, :], axis=1,
                         dtype=jnp.bfloat16).astype(jnp.float32)
            sc = sc + 1.5 * (ar[d0:d0 + DBLK] + al)
            rows = d0 + jax.lax.broadcasted_iota(jnp.int32, (DBLK, N), 0)
            cols = jax.lax.broadcasted_iota(jnp.int32, (DBLK, N), 1)
            sc = jnp.where(rows == cols, -1e30, sc)
            m = jnp.max(sc, axis=1, keepdims=True)
            e = jnp.exp(sc - m)
            den = jnp.sum(e, axis=1, keepdims=True)
            a = e / (den + 1e-16)
            blocks.append(jnp.dot(a.astype(jnp.bfloat16), xl_h.astype(jnp.bfloat16),
                                  preferred_element_type=jnp.float32))
        outs.append(jnp.concatenate(blocks, axis=0) if len(blocks) > 1 else blocks[0])
    return jnp.concatenate(outs, axis=1)


def _body(x_ref, wlt1_ref, bl1_ref, wrt1_ref, br1_ref, att1_ref, bias1_ref,
          wlt2_hbm, bl2_ref, wrt2_hbm, br2_ref, att2_ref, bias2_ref,
          wot_hbm, bo_ref, lng_ref, lnb_ref, resw_ref, out_ref,
          wlt2_ref, wrt2_ref, wot_ref, sem):
    cp2l = pltpu.make_async_copy(wlt2_hbm, wlt2_ref, sem.at[0])
    cp2r = pltpu.make_async_copy(wrt2_hbm, wrt2_ref, sem.at[1])
    cpo = pltpu.make_async_copy(wot_hbm, wot_ref, sem.at[2])
    cp2l.start()
    cp2r.start()
    cpo.start()

    x = x_ref[...]
    xb = x.astype(jnp.bfloat16)
    xl1 = jnp.dot(xb, wlt1_ref[...], preferred_element_type=jnp.float32) + bl1_ref[...]
    xr1 = jnp.dot(xb, wrt1_ref[...], preferred_element_type=jnp.float32) + br1_ref[...]
    o1 = _attn(xl1, xr1, att1_ref, N, HC1) + bias1_ref[...]
    y = jnp.where(o1 > 0, o1, jnp.exp(jnp.minimum(o1, 0.0)) - 1.0)

    cp2l.wait()
    cp2r.wait()
    yb = y.astype(jnp.bfloat16)
    xl2 = jnp.dot(yb, wlt2_ref[...], preferred_element_type=jnp.float32) + bl2_ref[...]
    xr2 = jnp.dot(yb[:NOUT], wrt2_ref[...], preferred_element_type=jnp.float32) + br2_ref[...]
    o2 = _attn(xl2, xr2, att2_ref, NOUT, HC2) + bias2_ref[...]

    cpo.wait()
    h = jnp.dot(o2, wot_ref[...], preferred_element_type=jnp.float32) + bo_ref[...]
    mu = jnp.mean(h, axis=-1, keepdims=True)
    var = jnp.mean((h - mu) ** 2, axis=-1, keepdims=True)
    h = (h - mu) * jax.lax.rsqrt(var + 1e-5) * lng_ref[...] + lnb_ref[...]
    h = jnp.maximum(h, 0.0)
    out_ref[...] = h + resw_ref[0, 0] * x[:NOUT]


def kernel(video_1_fea, video_2_fea, video_1_fused, video_2_fused,
           Wl1, bl1, Wr1, br1, att1, bias1,
           Wl2, bl2, Wr2, br2, att2, bias2,
           Wo, bo, ln_g, ln_b, res_w):
    B1, T, C = video_1_fea.shape
    B2 = video_2_fea.shape[0]
    x = jnp.concatenate([
        video_1_fea.reshape(B1 * T, C),
        video_2_fea.reshape(B2 * T, C),
        video_1_fused.reshape(B1 * T, C),
        video_2_fused.reshape(B2 * T, C),
    ], axis=0)

    f32 = jnp.float32
    bf16 = jnp.bfloat16
    vmem = pl.BlockSpec(memory_space=pltpu.VMEM)
    hbm = pl.BlockSpec(memory_space=pltpu.MemorySpace.HBM)
    h = pl.pallas_call(
        _body,
        in_specs=[vmem, vmem, vmem, vmem, vmem, vmem, vmem,
                  hbm, vmem, hbm, vmem, vmem, vmem,
                  hbm, vmem, vmem, vmem, vmem],
        scratch_shapes=[
            pltpu.VMEM((HEADS * HC1, HEADS * HC2), bf16),
            pltpu.VMEM((HEADS * HC1, HEADS * HC2), bf16),
            pltpu.VMEM((HEADS * HC2, C), f32),
            pltpu.SemaphoreType.DMA((3,)),
        ],
        out_shape=jax.ShapeDtypeStruct((NOUT, C), f32),
    )(x, Wl1.T.astype(jnp.bfloat16), bl1.reshape(1, -1),
      Wr1.T.astype(jnp.bfloat16), br1.reshape(1, -1), att1, bias1.reshape(1, -1),
      Wl2.T.astype(jnp.bfloat16), bl2.reshape(1, -1),
      Wr2.T.astype(jnp.bfloat16), br2.reshape(1, -1), att2, bias2.reshape(1, -1),
      Wo.T, bo.reshape(1, -1), ln_g.reshape(1, -1), ln_b.reshape(1, -1),
      res_w.reshape(1, 1))

    p1 = h[:B1 * T].reshape(B1, T, C)
    p2 = h[B1 * T:].reshape(B2, T, C)
    return jnp.concatenate([p1, p2], axis=0)


# speedup vs baseline: 1.0896x; 1.0738x over previous
import jax
import jax.numpy as jnp
from jax.experimental import pallas as pl
from jax.experimental.pallas import tpu as pltpu

N = 256
NOUT = 128
HEADS = 4
HC1 = 256
HC2 = 128
DBLK = 32


def _attn(xl, xr, att_ref, nd, hc):
    outs = []
    for h in range(HEADS):
        xl_h = xl[:, h * hc:(h + 1) * hc]
        att4 = 0.4 * att_ref[h:h + 1, :]
        xlTf = jnp.transpose(xl_h * att4)
        xlT = xlTf.astype(jnp.bfloat16)
        sgn = jnp.broadcast_to(jnp.sign(jnp.transpose(att4)), (hc, N)).astype(jnp.bfloat16)
        al = jnp.sum(xlTf, axis=0, keepdims=True)
        xr_h = xr[:, h * hc:(h + 1) * hc] * att4
        ar = jnp.sum(xr_h, axis=1, keepdims=True)
        xr16 = xr_h.astype(jnp.bfloat16)

        blocks = []
        for b in range(nd // DBLK):
            d0 = b * DBLK
            xrb = xr16[d0:d0 + DBLK]
            t = xrb[:, :, None] + xlT[None, :, :]
            sc = jnp.sum(jnp.abs(t) * sgn[None, :, :], axis=1,
                         dtype=jnp.bfloat16).astype(jnp.float32)
            sc = sc + 1.5 * (ar[d0:d0 + DBLK] + al)
            rows = d0 + jax.lax.broadcasted_iota(jnp.int32, (DBLK, N), 0)
            cols = jax.lax.broadcasted_iota(jnp.int32, (DBLK, N), 1)
            sc = jnp.where(rows == cols, -1e30, sc)
            m = jnp.max(sc, axis=1, keepdims=True)
            e = jnp.exp(sc - m)
            den = jnp.sum(e, axis=1, keepdims=True)
            a = e / (den + 1e-16)
            blocks.append(jnp.dot(a.astype(jnp.bfloat16), xl_h.astype(jnp.bfloat16),
                                  preferred_element_type=jnp.float32))
        outs.append(jnp.concatenate(blocks, axis=0) if len(blocks) > 1 else blocks[0])
    return jnp.concatenate(outs, axis=1)


def _body(v1_ref, v2_ref, v3_ref, v4_ref,
          wlt1_ref, bl1_ref, wrt1_ref, br1_ref, att1_ref, bias1_ref,
          wlt2_hbm, bl2_ref, wrt2_hbm, br2_ref, att2_ref, bias2_ref,
          wot_hbm, bo_ref, lng_ref, lnb_ref, resw_ref, out_ref,
          wlt2_ref, wrt2_ref, wot_ref, sem):
    cp2l = pltpu.make_async_copy(wlt2_hbm, wlt2_ref, sem.at[0])
    cp2r = pltpu.make_async_copy(wrt2_hbm, wrt2_ref, sem.at[1])
    cpo = pltpu.make_async_copy(wot_hbm, wot_ref, sem.at[2])
    cp2l.start()
    cp2r.start()
    cpo.start()

    x = jnp.concatenate([v1_ref[...], v2_ref[...], v3_ref[...], v4_ref[...]], axis=0)
    xb = x.astype(jnp.bfloat16)
    xl1 = jnp.dot(xb, wlt1_ref[...], preferred_element_type=jnp.float32) + bl1_ref[...]
    xr1 = jnp.dot(xb, wrt1_ref[...], preferred_element_type=jnp.float32) + br1_ref[...]
    o1 = _attn(xl1, xr1, att1_ref, N, HC1) + bias1_ref[...]
    y = jnp.where(o1 > 0, o1, jnp.exp(jnp.minimum(o1, 0.0)) - 1.0)

    cp2l.wait()
    cp2r.wait()
    yb = y.astype(jnp.bfloat16)
    xl2 = jnp.dot(yb, wlt2_ref[...], preferred_element_type=jnp.float32) + bl2_ref[...]
    xr2 = jnp.dot(yb[:NOUT], wrt2_ref[...], preferred_element_type=jnp.float32) + br2_ref[...]
    o2 = _attn(xl2, xr2, att2_ref, NOUT, HC2) + bias2_ref[...]

    cpo.wait()
    h = jnp.dot(o2, wot_ref[...], preferred_element_type=jnp.float32) + bo_ref[...]
    mu = jnp.mean(h, axis=-1, keepdims=True)
    var = jnp.mean((h - mu) ** 2, axis=-1, keepdims=True)
    h = (h - mu) * jax.lax.rsqrt(var + 1e-5) * lng_ref[...] + lnb_ref[...]
    h = jnp.maximum(h, 0.0)
    out_ref[...] = h + resw_ref[0, 0] * x[:NOUT]


def kernel(video_1_fea, video_2_fea, video_1_fused, video_2_fused,
           Wl1, bl1, Wr1, br1, att1, bias1,
           Wl2, bl2, Wr2, br2, att2, bias2,
           Wo, bo, ln_g, ln_b, res_w):
    B1, T, C = video_1_fea.shape
    B2 = video_2_fea.shape[0]
    f32 = jnp.float32
    bf16 = jnp.bfloat16
    vmem = pl.BlockSpec(memory_space=pltpu.VMEM)
    hbm = pl.BlockSpec(memory_space=pltpu.MemorySpace.HBM)
    h = pl.pallas_call(
        _body,
        in_specs=[vmem, vmem, vmem, vmem,
                  vmem, vmem, vmem, vmem, vmem, vmem,
                  hbm, vmem, hbm, vmem, vmem, vmem,
                  hbm, vmem, vmem, vmem, vmem],
        scratch_shapes=[
            pltpu.VMEM((HEADS * HC1, HEADS * HC2), bf16),
            pltpu.VMEM((HEADS * HC1, HEADS * HC2), bf16),
            pltpu.VMEM((HEADS * HC2, C), f32),
            pltpu.SemaphoreType.DMA((3,)),
        ],
        out_shape=jax.ShapeDtypeStruct((NOUT, C), f32),
    )(video_1_fea.reshape(B1 * T, C), video_2_fea.reshape(B2 * T, C),
      video_1_fused.reshape(B1 * T, C), video_2_fused.reshape(B2 * T, C),
      Wl1.T.astype(jnp.bfloat16), bl1.reshape(1, -1),
      Wr1.T.astype(jnp.bfloat16), br1.reshape(1, -1), att1, bias1.reshape(1, -1),
      Wl2.T.astype(jnp.bfloat16), bl2.reshape(1, -1),
      Wr2.T.astype(jnp.bfloat16), br2.reshape(1, -1), att2, bias2.reshape(1, -1),
      Wo.T, bo.reshape(1, -1), ln_g.reshape(1, -1), ln_b.reshape(1, -1),
      res_w.reshape(1, 1))

    return h.reshape(B1 + B2, T, C)


# speedup vs baseline: 1.1674x; 1.0714x over previous
import jax
import jax.numpy as jnp
from jax.experimental import pallas as pl
from jax.experimental.pallas import tpu as pltpu

N = 256
NOUT = 128
HEADS = 4
HC1 = 256
HC2 = 128
DBLK = 32


def _attn(xl, xr, att_ref, nd, hc):
    outs = []
    for h in range(HEADS):
        xl_h = xl[:, h * hc:(h + 1) * hc]
        att4 = 0.4 * att_ref[h:h + 1, :]
        xlTf = jnp.transpose(xl_h * att4)
        xlT = xlTf.astype(jnp.bfloat16)
        sgn = jnp.broadcast_to(jnp.sign(jnp.transpose(att4)), (hc, N)).astype(jnp.bfloat16)
        al = jnp.sum(xlTf, axis=0, keepdims=True)
        xr_h = xr[:, h * hc:(h + 1) * hc] * att4
        ar = jnp.sum(xr_h, axis=1, keepdims=True)
        xr16 = xr_h.astype(jnp.bfloat16)

        blocks = []
        for b in range(nd // DBLK):
            d0 = b * DBLK
            xrb = xr16[d0:d0 + DBLK]
            t = xrb[:, :, None] + xlT[None, :, :]
            sc = jnp.sum(jnp.abs(t) * sgn[None, :, :], axis=1,
                         dtype=jnp.bfloat16).astype(jnp.float32)
            sc = sc + 1.5 * (ar[d0:d0 + DBLK] + al)
            rows = d0 + jax.lax.broadcasted_iota(jnp.int32, (DBLK, N), 0)
            cols = jax.lax.broadcasted_iota(jnp.int32, (DBLK, N), 1)
            sc = jnp.where(rows == cols, -1e30, sc)
            m = jnp.max(sc, axis=1, keepdims=True)
            e = jnp.exp(sc - m)
            den = jnp.sum(e, axis=1, keepdims=True)
            a = e / (den + 1e-16)
            blocks.append(jnp.dot(a.astype(jnp.bfloat16), xl_h.astype(jnp.bfloat16),
                                  preferred_element_type=jnp.float32))
        outs.append(jnp.concatenate(blocks, axis=0) if len(blocks) > 1 else blocks[0])
    return jnp.concatenate(outs, axis=1)


def _body(v1_ref, v2_ref, v3_ref, v4_ref,
          wlt1_ref, bl1_ref, wrt1_ref, br1_ref, att1_ref, bias1_ref,
          wlt2_hbm, bl2_ref, wrt2_hbm, br2_ref, att2_ref, bias2_ref,
          wot_hbm, bo_ref, lng_ref, lnb_ref, resw_ref, out_ref,
          wlt2_ref, wrt2_ref, wot_ref, sem):
    cp2l = pltpu.make_async_copy(wlt2_hbm, wlt2_ref, sem.at[0])
    cp2r = pltpu.make_async_copy(wrt2_hbm, wrt2_ref, sem.at[1])
    cpo = pltpu.make_async_copy(wot_hbm, wot_ref, sem.at[2])
    cp2l.start()
    cp2r.start()
    cpo.start()

    dnT = (((1,), (1,)), ((), ()))
    x = jnp.concatenate([v1_ref[...], v2_ref[...], v3_ref[...], v4_ref[...]], axis=0)
    xb = x.astype(jnp.bfloat16)
    xl1 = jax.lax.dot_general(xb, wlt1_ref[...].astype(jnp.bfloat16), dnT,
                              preferred_element_type=jnp.float32) + bl1_ref[...]
    xr1 = jax.lax.dot_general(xb, wrt1_ref[...].astype(jnp.bfloat16), dnT,
                              preferred_element_type=jnp.float32) + br1_ref[...]
    o1 = _attn(xl1, xr1, att1_ref, N, HC1) + bias1_ref[...]
    y = jnp.where(o1 > 0, o1, jnp.exp(jnp.minimum(o1, 0.0)) - 1.0)

    cp2l.wait()
    cp2r.wait()
    yb = y.astype(jnp.bfloat16)
    xl2 = jax.lax.dot_general(yb, wlt2_ref[...].astype(jnp.bfloat16), dnT,
                              preferred_element_type=jnp.float32) + bl2_ref[...]
    xr2 = jax.lax.dot_general(yb[:NOUT], wrt2_ref[...].astype(jnp.bfloat16), dnT,
                              preferred_element_type=jnp.float32) + br2_ref[...]
    o2 = _attn(xl2, xr2, att2_ref, NOUT, HC2) + bias2_ref[...]

    cpo.wait()
    h = jax.lax.dot_general(o2, wot_ref[...], dnT,
                            preferred_element_type=jnp.float32) + bo_ref[...]
    mu = jnp.mean(h, axis=-1, keepdims=True)
    var = jnp.mean((h - mu) ** 2, axis=-1, keepdims=True)
    h = (h - mu) * jax.lax.rsqrt(var + 1e-5) * lng_ref[...] + lnb_ref[...]
    h = jnp.maximum(h, 0.0)
    out_ref[...] = h + resw_ref[0, 0] * x[:NOUT]


def kernel(video_1_fea, video_2_fea, video_1_fused, video_2_fused,
           Wl1, bl1, Wr1, br1, att1, bias1,
           Wl2, bl2, Wr2, br2, att2, bias2,
           Wo, bo, ln_g, ln_b, res_w):
    B1, T, C = video_1_fea.shape
    B2 = video_2_fea.shape[0]
    f32 = jnp.float32
    bf16 = jnp.bfloat16
    vmem = pl.BlockSpec(memory_space=pltpu.VMEM)
    hbm = pl.BlockSpec(memory_space=pltpu.MemorySpace.HBM)
    h = pl.pallas_call(
        _body,
        in_specs=[vmem, vmem, vmem, vmem,
                  vmem, vmem, vmem, vmem, vmem, vmem,
                  hbm, vmem, hbm, vmem, vmem, vmem,
                  hbm, vmem, vmem, vmem, vmem],
        scratch_shapes=[
            pltpu.VMEM((HEADS * HC2, HEADS * HC1), f32),
            pltpu.VMEM((HEADS * HC2, HEADS * HC1), f32),
            pltpu.VMEM((C, C), f32),
            pltpu.SemaphoreType.DMA((3,)),
        ],
        out_shape=jax.ShapeDtypeStruct((NOUT, C), f32),
    )(video_1_fea.reshape(B1 * T, C), video_2_fea.reshape(B2 * T, C),
      video_1_fused.reshape(B1 * T, C), video_2_fused.reshape(B2 * T, C),
      Wl1, bl1.reshape(1, -1),
      Wr1, br1.reshape(1, -1), att1, bias1.reshape(1, -1),
      Wl2, bl2.reshape(1, -1),
      Wr2, br2.reshape(1, -1), att2, bias2.reshape(1, -1),
      Wo, bo.reshape(1, -1), ln_g.reshape(1, -1), ln_b.reshape(1, -1),
      res_w.reshape(1, 1))

    return h.reshape(B1 + B2, T, C)


# speedup vs baseline: 1.1707x; 1.0029x over previous
import jax
import jax.numpy as jnp
from jax.experimental import pallas as pl
from jax.experimental.pallas import tpu as pltpu

N = 256
NOUT = 128
HEADS = 4
HC1 = 256
HC2 = 128
DBLK = 32


def _attn(xl, xr, att_ref, nd, hc):
    outs = []
    for h in range(HEADS):
        xl_h = xl[:, h * hc:(h + 1) * hc]
        att4 = 0.4 * att_ref[h:h + 1, :]
        xlTf = jnp.transpose(xl_h * att4)
        xlT = xlTf.astype(jnp.bfloat16)
        sgn = jnp.broadcast_to(jnp.sign(jnp.transpose(att4)), (hc, N)).astype(jnp.bfloat16)
        al = jnp.sum(xlTf, axis=0, keepdims=True)
        xr_h = xr[:, h * hc:(h + 1) * hc] * att4
        ar = jnp.sum(xr_h, axis=1, keepdims=True)
        xr16 = xr_h.astype(jnp.bfloat16)

        blocks = []
        for b in range(nd // DBLK):
            d0 = b * DBLK
            xrb = xr16[d0:d0 + DBLK]
            t = xrb[:, :, None] + xlT[None, :, :]
            sc = jnp.sum(jnp.abs(t) * sgn[None, :, :], axis=1,
                         dtype=jnp.bfloat16).astype(jnp.float32)
            sc = sc + 1.5 * (ar[d0:d0 + DBLK] + al)
            rows = d0 + jax.lax.broadcasted_iota(jnp.int32, (DBLK, N), 0)
            cols = jax.lax.broadcasted_iota(jnp.int32, (DBLK, N), 1)
            sc = jnp.where(rows == cols, -1e30, sc)
            m = jnp.max(sc, axis=1, keepdims=True)
            e = jnp.exp(sc - m)
            den = jnp.sum(e, axis=1, keepdims=True)
            a = e / (den + 1e-16)
            blocks.append(jnp.dot(a.astype(jnp.bfloat16), xl_h.astype(jnp.bfloat16),
                                  preferred_element_type=jnp.float32))
        outs.append(jnp.concatenate(blocks, axis=0) if len(blocks) > 1 else blocks[0])
    return jnp.concatenate(outs, axis=1)


def _body(v1_ref, v2_ref, v3_ref, v4_ref,
          wl1_ref, bl1_ref, wr1_ref, br1_ref, att1_ref, bias1_ref,
          wl2_hbm, bl2_ref, wr2_hbm, br2_ref, att2_ref, bias2_ref,
          wo_hbm, bo_ref, lng_ref, lnb_ref, resw_ref, out_ref,
          wl2_ref, wr2_ref, wo_ref, sem):
    cp2l = pltpu.make_async_copy(wl2_hbm, wl2_ref, sem.at[0])
    cp2r = pltpu.make_async_copy(wr2_hbm, wr2_ref, sem.at[1])
    cpo = pltpu.make_async_copy(wo_hbm, wo_ref, sem.at[2])
    cp2l.start()
    cp2r.start()
    cpo.start()

    dnT = (((1,), (1,)), ((), ()))
    x = jnp.concatenate([v1_ref[...], v2_ref[...], v3_ref[...], v4_ref[...]], axis=0)
    xb = x.astype(jnp.bfloat16)
    xl1 = jax.lax.dot_general(xb, wl1_ref[...].astype(jnp.bfloat16), dnT,
                              preferred_element_type=jnp.float32) + bl1_ref[...]
    xr1 = jax.lax.dot_general(xb, wr1_ref[...].astype(jnp.bfloat16), dnT,
                              preferred_element_type=jnp.float32) + br1_ref[...]
    o1 = _attn(xl1, xr1, att1_ref, N, HC1) + bias1_ref[...]
    y = jnp.where(o1 > 0, o1, jnp.exp(jnp.minimum(o1, 0.0)) - 1.0)

    cp2l.wait()
    cp2r.wait()
    yb = y.astype(jnp.bfloat16)
    xl2 = jax.lax.dot_general(yb, wl2_ref[...].astype(jnp.bfloat16), dnT,
                              preferred_element_type=jnp.float32) + bl2_ref[...]
    xr2 = jax.lax.dot_general(yb[:NOUT], wr2_ref[...].astype(jnp.bfloat16), dnT,
                              preferred_element_type=jnp.float32) + br2_ref[...]
    o2 = _attn(xl2, xr2, att2_ref, NOUT, HC2) + bias2_ref[...]

    cpo.wait()
    h = jax.lax.dot_general(o2, wo_ref[...], dnT,
                            preferred_element_type=jnp.float32) + bo_ref[...]
    mu = jnp.mean(h, axis=-1, keepdims=True)
    var = jnp.mean((h - mu) ** 2, axis=-1, keepdims=True)
    h = (h - mu) * jax.lax.rsqrt(var + 1e-5) * lng_ref[...] + lnb_ref[...]
    h = jnp.maximum(h, 0.0)
    out_ref[...] = h + resw_ref[0, 0] * x[:NOUT]


def kernel(video_1_fea, video_2_fea, video_1_fused, video_2_fused,
           Wl1, bl1, Wr1, br1, att1, bias1,
           Wl2, bl2, Wr2, br2, att2, bias2,
           Wo, bo, ln_g, ln_b, res_w):
    B1, T, C = video_1_fea.shape
    B2 = video_2_fea.shape[0]
    f32 = jnp.float32
    bf16 = jnp.bfloat16
    vmem = pl.BlockSpec(memory_space=pltpu.VMEM)
    hbm = pl.BlockSpec(memory_space=pltpu.MemorySpace.HBM)
    h = pl.pallas_call(
        _body,
        in_specs=[vmem, vmem, vmem, vmem,
                  vmem, vmem, vmem, vmem, vmem, vmem,
                  hbm, vmem, hbm, vmem, vmem, vmem,
                  hbm, vmem, vmem, vmem, vmem],
        scratch_shapes=[
            pltpu.VMEM((HEADS * HC2, HEADS * HC1), f32),
            pltpu.VMEM((HEADS * HC2, HEADS * HC1), f32),
            pltpu.VMEM((C, C), f32),
            pltpu.SemaphoreType.DMA((3,)),
        ],
        out_shape=jax.ShapeDtypeStruct((NOUT, C), f32),
    )(video_1_fea.reshape(B1 * T, C), video_2_fea.reshape(B2 * T, C),
      video_1_fused.reshape(B1 * T, C), video_2_fused.reshape(B2 * T, C),
      Wl1, bl1.reshape(1, -1),
      Wr1, br1.reshape(1, -1), att1, bias1.reshape(1, -1),
      Wl2, bl2.reshape(1, -1),
      Wr2, br2.reshape(1, -1), att2, bias2.reshape(1, -1),
      Wo, bo.reshape(1, -1), ln_g.reshape(1, -1), ln_b.reshape(1, -1),
      res_w.reshape(1, 1))

    return h.reshape(B1 + B2, T, C)
